# Initial kernel scaffold; baseline (speedup 1.0000x reference)
import functools

import numpy as np
import jax
import jax.numpy as jnp
from jax import lax
from jax.experimental import pallas as pl
from jax.experimental.pallas import tpu as pltpu
from jax.experimental.pallas import tpu_sc as plsc

_N = 4096
_D = 256
_T = 3
_G = 64
_NW = 32
_CH = _N // _NW

_ts = np.arange(_N, dtype=np.int64)
_f1 = ((_ts - 1) % _N) * _N + _ts
_f2 = ((_ts + 1) % _N) * _N + _ts
_ROWS1 = (_f1 // _G).astype(np.int32)
_COLS1 = (_f1 % _G).astype(np.int32)
_ROWS2 = (_f2 // _G).astype(np.int32)
_COLS2 = (_f2 % _G).astype(np.int32)


def _edge_gather_body(ef_rows, rows1, cols1, rows2, cols2, e1_out, e2_out,
                      idx_v, cols_v, seg_v, out_v, sem):
    wid = lax.axis_index("s") * 2 + lax.axis_index("c")
    base = wid * _CH
    for rows_hbm, cols_hbm, out_hbm in ((rows1, cols1, e1_out),
                                        (rows2, cols2, e2_out)):
        pltpu.sync_copy(rows_hbm.at[pl.ds(base, _CH)], idx_v)
        pltpu.async_copy(ef_rows.at[idx_v], seg_v, sem).wait()
        pltpu.sync_copy(cols_hbm.at[pl.ds(base, _CH)], cols_v)
        for j in range(_CH // 16):
            lidx = lax.iota(jnp.int32, 16) + j * 16
            c = cols_v[pl.ds(j * 16, 16)]
            out_v[pl.ds(j * 16, 16)] = plsc.load_gather(seg_v, [lidx, c])
        pltpu.sync_copy(out_v, out_hbm.at[pl.ds(base, _CH)])


_edge_gather = functools.partial(
    pl.kernel,
    out_type=(jax.ShapeDtypeStruct((_N,), jnp.float32),
              jax.ShapeDtypeStruct((_N,), jnp.float32)),
    mesh=plsc.VectorSubcoreMesh(core_axis_name="c", subcore_axis_name="s"),
    scratch_types=[
        pltpu.VMEM((_CH,), jnp.int32),
        pltpu.VMEM((_CH,), jnp.int32),
        pltpu.VMEM((_CH, _G), jnp.float32),
        pltpu.VMEM((_CH,), jnp.float32),
        pltpu.SemaphoreType.DMA,
    ],
)(_edge_gather_body)


def _mpnn_body(nf_ref, e1_ref, e2_ref, ws_ref, wt_ref, we_ref, bm_ref,
               wih_ref, whh_ref, bih_ref, bhh_ref, wpol_ref, bpol_ref,
               out_ref):
    f32 = jnp.float32
    nf = nf_ref[...]
    E1 = e1_ref[...] * we_ref[...] + bm_ref[...]
    E2 = e2_ref[...] * we_ref[...] + bm_ref[...]
    ws = ws_ref[...]
    wt = wt_ref[...]
    wih = wih_ref[...]
    whh = whh_ref[...]
    bih = bih_ref[...]
    bhh = bhh_ref[...]
    for _ in range(_T):
        p = jnp.dot(nf, ws, preferred_element_type=f32)
        q = jnp.dot(nf, wt, preferred_element_type=f32)
        pm = pltpu.roll(p, 1, axis=0)
        pp = pltpu.roll(p, -1, axis=0)
        agg = (jnp.maximum(pm + q + E1, 0.0) +
               jnp.maximum(pp + q + E2, 0.0))
        gi = jnp.dot(agg, wih, preferred_element_type=f32) + bih
        gh = jnp.dot(nf, whh, preferred_element_type=f32) + bhh
        r = jax.nn.sigmoid(gi[:, :_D] + gh[:, :_D])
        z = jax.nn.sigmoid(gi[:, _D:2 * _D] + gh[:, _D:2 * _D])
        n = jnp.tanh(gi[:, 2 * _D:] + r * gh[:, 2 * _D:])
        nf = (1.0 - z) * n + z * nf
    s = jnp.sum(nf, axis=0, keepdims=True)
    out_ref[...] = (jnp.dot(s, wpol_ref[...], preferred_element_type=f32)
                    + bpol_ref[...])


def kernel(node_features, edge_features, adjacency_matrix, W_msg, b_msg,
           W_ih, W_hh, b_ih, b_hh, W_pol, b_pol):
    del adjacency_matrix
    a_dim = W_pol.shape[0]
    ef_rows = edge_features.reshape(_N * _N // _G, _G)
    e1, e2 = _edge_gather(ef_rows,
                          jnp.asarray(_ROWS1), jnp.asarray(_COLS1),
                          jnp.asarray(_ROWS2), jnp.asarray(_COLS2))
    out = pl.pallas_call(
        _mpnn_body,
        out_shape=jax.ShapeDtypeStruct((1, a_dim), jnp.float32),
    )(
        node_features[0],
        e1[:, None], e2[:, None],
        W_msg[:, :_D].T, W_msg[:, _D:2 * _D].T,
        W_msg[:, 2 * _D][None, :], b_msg[None, :],
        W_ih.T, W_hh.T, b_ih[None, :], b_hh[None, :],
        W_pol.T, b_pol[None, :],
    )
    return out


# speedup vs baseline: 111.4440x; 111.4440x over previous
import functools

import numpy as np
import jax
import jax.numpy as jnp
from jax import lax
from jax.experimental import pallas as pl
from jax.experimental.pallas import tpu as pltpu
from jax.experimental.pallas import tpu_sc as plsc

_N = 4096
_D = 256
_T = 3
_NW = 32
_CH = _N // _NW

_LANES = 128
_ts = np.arange(_N, dtype=np.int64)
_F1 = ((_ts - 1) % _N) * _N + _ts
_F2 = ((_ts + 1) % _N) * _N + _ts
_R1 = (_F1 // _LANES).astype(np.int32)
_R2 = (_F2 // _LANES).astype(np.int32)
assert np.all(_F1 % _LANES == _ts % _LANES) and np.all(_F2 % _LANES == _ts % _LANES)


def _edge_gather_body(ef_rows, idx1, idx2, e1_out, e2_out, idx_v, seg_v, sem):
    wid = lax.axis_index("s") * 2 + lax.axis_index("c")
    base = wid * _CH
    for idx_hbm, out_hbm in ((idx1, e1_out), (idx2, e2_out)):
        pltpu.sync_copy(idx_hbm.at[pl.ds(base, _CH)], idx_v)
        pltpu.async_copy(ef_rows.at[idx_v], seg_v, sem).wait()
        pltpu.sync_copy(seg_v, out_hbm.at[pl.ds(base, _CH)])


@functools.cache
def _edge_gather():
    return pl.kernel(
        _edge_gather_body,
        out_type=(jax.ShapeDtypeStruct((_N, _LANES), jnp.float32),
                  jax.ShapeDtypeStruct((_N, _LANES), jnp.float32)),
        mesh=plsc.VectorSubcoreMesh(core_axis_name="c", subcore_axis_name="s"),
        scratch_types=[
            pltpu.VMEM((_CH,), jnp.int32),
            pltpu.VMEM((_CH, _LANES), jnp.float32),
            pltpu.SemaphoreType.DMA,
        ],
    )


def _mpnn_body(nf_ref, e1_ref, e2_ref, ws_ref, wt_ref, we_ref, bm_ref,
               wih_ref, whh_ref, bih_ref, bhh_ref, wpol_ref, bpol_ref,
               out_ref):
    f32 = jnp.float32
    nf = nf_ref[...]
    lane = lax.broadcasted_iota(jnp.int32, (_N, _LANES), 1)
    row = lax.broadcasted_iota(jnp.int32, (_N, _LANES), 0)
    dmask = (lane == (row & (_LANES - 1))).astype(f32)
    e1 = jnp.sum(e1_ref[...] * dmask, axis=1, keepdims=True)
    e2 = jnp.sum(e2_ref[...] * dmask, axis=1, keepdims=True)
    E1 = e1 * we_ref[...] + bm_ref[...]
    E2 = e2 * we_ref[...] + bm_ref[...]
    ws = ws_ref[...]
    wt = wt_ref[...]
    wih = wih_ref[...]
    whh = whh_ref[...]
    bih = bih_ref[...]
    bhh = bhh_ref[...]
    for _ in range(_T):
        p = jnp.dot(nf, ws, preferred_element_type=f32)
        q = jnp.dot(nf, wt, preferred_element_type=f32)
        pm = pltpu.roll(p, 1, axis=0)
        pp = pltpu.roll(p, _N - 1, axis=0)
        agg = (jnp.maximum(pm + q + E1, 0.0) +
               jnp.maximum(pp + q + E2, 0.0))
        gi = jnp.dot(agg, wih, preferred_element_type=f32) + bih
        gh = jnp.dot(nf, whh, preferred_element_type=f32) + bhh
        r = jax.nn.sigmoid(gi[:, :_D] + gh[:, :_D])
        z = jax.nn.sigmoid(gi[:, _D:2 * _D] + gh[:, _D:2 * _D])
        n = jnp.tanh(gi[:, 2 * _D:] + r * gh[:, 2 * _D:])
        nf = (1.0 - z) * n + z * nf
    s = jnp.sum(nf, axis=0, keepdims=True)
    out_ref[...] = (jnp.dot(s, wpol_ref[...], preferred_element_type=f32)
                    + bpol_ref[...])


def kernel(node_features, edge_features, adjacency_matrix, W_msg, b_msg,
           W_ih, W_hh, b_ih, b_hh, W_pol, b_pol):
    del adjacency_matrix
    a_dim = W_pol.shape[0]
    ef_rows = edge_features.reshape(_N * _N // _LANES, _LANES)
    e1, e2 = _edge_gather()(ef_rows, jnp.asarray(_R1), jnp.asarray(_R2))
    out = pl.pallas_call(
        _mpnn_body,
        out_shape=jax.ShapeDtypeStruct((1, a_dim), jnp.float32),
    )(
        node_features[0],
        e1, e2,
        W_msg[:, :_D].T, W_msg[:, _D:2 * _D].T,
        W_msg[:, 2 * _D][None, :], b_msg[None, :],
        W_ih.T, W_hh.T, b_ih[None, :], b_hh[None, :],
        W_pol.T, b_pol[None, :],
    )
    return out


# speedup vs baseline: 287.9982x; 2.5842x over previous
import functools

import jax
import jax.numpy as jnp
from jax import lax
from jax.experimental import pallas as pl
from jax.experimental.pallas import tpu as pltpu
from jax.experimental.pallas import tpu_sc as plsc

_N = 4096
_D = 256
_T = 3
_LANES = 128
_NW = 32
_CH = _N // _NW


_BAND = _CH + 16


def _edge_gather_body(ef, e1_out, e2_out, band_v, sem):
    wid = lax.axis_index("s") * 2 + lax.axis_index("c")
    base = wid * _CH
    @pl.when(wid == 0)
    def _():
        pltpu.sync_copy(ef.at[0, pl.ds(_N - 8, 8), pl.ds(0, _LANES)],
                        band_v.at[pl.ds(0, 8)])
        pltpu.sync_copy(ef.at[0, pl.ds(0, _BAND - 8), pl.ds(0, _LANES)],
                        band_v.at[pl.ds(8, _BAND - 8)])

    @pl.when(wid == _NW - 1)
    def _():
        pltpu.sync_copy(ef.at[0, pl.ds(_N - _CH - 8, _BAND - 8), pl.ds(base, _LANES)],
                        band_v.at[pl.ds(0, _BAND - 8)])
        pltpu.sync_copy(ef.at[0, pl.ds(0, 8), pl.ds(base, _LANES)],
                        band_v.at[pl.ds(_BAND - 8, 8)])

    @pl.when(jnp.logical_and(wid > 0, wid < _NW - 1))
    def _():
        pltpu.sync_copy(ef.at[0, pl.ds(base - 8, _BAND), pl.ds(base, _LANES)],
                        band_v)

    pltpu.sync_copy(band_v.at[pl.ds(7, _CH)], e1_out.at[pl.ds(base, _CH)])
    pltpu.sync_copy(band_v.at[pl.ds(9, _CH)], e2_out.at[pl.ds(base, _CH)])


@functools.cache
def _edge_gather():
    return pl.kernel(
        _edge_gather_body,
        out_type=(jax.ShapeDtypeStruct((_N, _LANES), jnp.float32),
                  jax.ShapeDtypeStruct((_N, _LANES), jnp.float32)),
        mesh=plsc.VectorSubcoreMesh(core_axis_name="c", subcore_axis_name="s"),
        scratch_types=[
            pltpu.VMEM((_BAND, _LANES), jnp.float32),
            pltpu.SemaphoreType.DMA,
        ],
    )


def _dot_t(x, w):
    return lax.dot_general(x, w, (((1,), (1,)), ((), ())),
                           preferred_element_type=jnp.float32)


def _mpnn_body(nf_ref, e1_ref, e2_ref, wmsg_ref, bm_ref, wih_ref, whh_ref,
               bih_ref, bhh_ref, wpol_ref, bpol_ref, out_ref):
    nf = nf_ref[...]
    lane = lax.broadcasted_iota(jnp.int32, (_N, _LANES), 1)
    row = lax.broadcasted_iota(jnp.int32, (_N, _LANES), 0)
    dmask = (lane == (row & (_LANES - 1))).astype(jnp.float32)
    e1 = jnp.sum(e1_ref[...] * dmask, axis=1, keepdims=True)
    e2 = jnp.sum(e2_ref[...] * dmask, axis=1, keepdims=True)
    wmsg = wmsg_ref[...]
    ws = wmsg[:, :_D]
    wt = wmsg[:, _D:2 * _D]
    we = wmsg[:, 2 * _D:]
    bm = bm_ref[...]
    E1 = _dot_t(e1, we) + bm
    E2 = _dot_t(e2, we) + bm
    wih = wih_ref[...]
    whh = whh_ref[...]
    bih = bih_ref[...]
    bhh = bhh_ref[...]
    for _ in range(_T):
        p = _dot_t(nf, ws)
        q = _dot_t(nf, wt)
        pm = pltpu.roll(p, 1, axis=0)
        pp = pltpu.roll(p, _N - 1, axis=0)
        agg = (jnp.maximum(pm + q + E1, 0.0) +
               jnp.maximum(pp + q + E2, 0.0))
        gi = _dot_t(agg, wih) + bih
        gh = _dot_t(nf, whh) + bhh
        r = jax.nn.sigmoid(gi[:, :_D] + gh[:, :_D])
        z = jax.nn.sigmoid(gi[:, _D:2 * _D] + gh[:, _D:2 * _D])
        n = jnp.tanh(gi[:, 2 * _D:] + r * gh[:, 2 * _D:])
        nf = (1.0 - z) * n + z * nf
    s = jnp.sum(nf, axis=0, keepdims=True)
    out_ref[...] = _dot_t(s, wpol_ref[...]) + bpol_ref[...]


def kernel(node_features, edge_features, adjacency_matrix, W_msg, b_msg,
           W_ih, W_hh, b_ih, b_hh, W_pol, b_pol):
    del adjacency_matrix
    a_dim = W_pol.shape[0]
    e1seg, e2seg = _edge_gather()(edge_features)
    out = pl.pallas_call(
        _mpnn_body,
        out_shape=jax.ShapeDtypeStruct((1, a_dim), jnp.float32),
    )(
        node_features[0],
        e1seg, e2seg,
        W_msg, b_msg[None, :],
        W_ih, W_hh, b_ih[None, :], b_hh[None, :],
        W_pol, b_pol[None, :],
    )
    return out
